# Initial kernel scaffold; baseline (speedup 1.0000x reference)
import jax
import jax.numpy as jnp
from jax.experimental import pallas as pl


def kernel(title_table, text_table, title_ids, token_ids):
    raise NotImplementedError("write your pallas kernel here")


# speedup vs baseline: 7.6394x; 7.6394x over previous
import functools

import jax
import jax.numpy as jnp
from jax import lax
from jax.experimental import pallas as pl
from jax.experimental.pallas import tpu as pltpu
from jax.experimental.pallas import tpu_sc as plsc

B = 16384
L = 20
D = 32
PK = 128

NC, NS = 2, 16
NW = NC * NS
SPW = B // NW
CH = 32
NCH = SPW // CH

_MESH = plsc.VectorSubcoreMesh(
    core_axis_name="c", subcore_axis_name="s", num_cores=NC, num_subcores=NS)


@functools.partial(
    pl.kernel,
    out_type=jax.ShapeDtypeStruct((B * 2 * D,), jnp.float32),
    mesh=_MESH,
    scratch_types=[
        pltpu.VMEM((L * CH,), jnp.int32),
        pltpu.VMEM((L * CH,), jnp.int32),
        pltpu.VMEM((L * CH, PK), jnp.float32),
        pltpu.VMEM((CH,), jnp.int32),
        pltpu.VMEM((CH,), jnp.int32),
        pltpu.VMEM((CH, PK), jnp.float32),
        pltpu.VMEM((CH * 2 * D,), jnp.float32),
        pltpu.VMEM((CH,), jnp.float32),
        pltpu.SemaphoreType.DMA,
        pltpu.SemaphoreType.DMA,
    ],
)
def _sc_kernel(title_hbm, text_hbm, tids_hbm, trids_hbm, tb_hbm, rb_hbm,
               out_hbm, ids_v, rid_v, rows_v, tidx_v, trid_v, trow_v,
               outb_v, recip_v, gsem, tsem):
    wid = lax.axis_index("s") * NC + lax.axis_index("c")

    def chunk(c, _):
        base = pl.multiple_of((wid * NCH + c) * CH, CH)
        pltpu.sync_copy(tb_hbm.at[pl.ds(base * L, L * CH)], ids_v)
        pltpu.sync_copy(rb_hbm.at[pl.ds(base * L, L * CH)], rid_v)
        pltpu.sync_copy(tids_hbm.at[pl.ds(base, CH)], tidx_v)
        pltpu.sync_copy(trids_hbm.at[pl.ds(base, CH)], trid_v)
        tcopy = pltpu.async_copy(title_hbm.at[trid_v], trow_v, tsem)
        gcopies = [
            pltpu.async_copy(text_hbm.at[rid_v.at[pl.ds(j * CH, CH)]],
                             rows_v.at[pl.ds(j * CH, CH)], gsem)
            for j in range(L)
        ]
        for s in range(CH // 16):
            cnt = jnp.zeros((16,), jnp.float32)
            for j in range(L):
                ids = ids_v[pl.ds(j * CH + s * 16, 16)]
                cnt = cnt + jnp.where(ids != 0, 1.0, 0.0)
            recip_v[pl.ds(s * 16, 16)] = 1.0 / jnp.maximum(cnt, 1.0)
        for gcp in gcopies:
            gcp.wait()
        tcopy.wait()

        for s in range(CH // 16):
            rvec = recip_v[pl.ds(s * 16, 16)]
            tqvec = (tidx_v[pl.ds(s * 16, 16)] & 3) * D
            qvecs = [(ids_v[pl.ds(j * CH + s * 16, 16)] & 3) * D
                     for j in range(L)]
            for i2 in range(16):
                i = s * 16 + i2
                q0 = qvecs[0][i2]
                a0 = rows_v[i, pl.ds(q0, 16)]
                a1 = rows_v[i, pl.ds(q0 + 16, 16)]
                for j in range(1, L):
                    qj = qvecs[j][i2]
                    a0 = a0 + rows_v[j * CH + i, pl.ds(qj, 16)]
                    a1 = a1 + rows_v[j * CH + i, pl.ds(qj + 16, 16)]
                r = rvec[i2]
                tq = tqvec[i2]
                outb_v[pl.ds(i * 2 * D, 16)] = trow_v[i, pl.ds(tq, 16)]
                outb_v[pl.ds(i * 2 * D + 16, 16)] = trow_v[i, pl.ds(tq + 16, 16)]
                outb_v[pl.ds(i * 2 * D + 32, 16)] = a0 * r
                outb_v[pl.ds(i * 2 * D + 48, 16)] = a1 * r
        pltpu.sync_copy(outb_v, out_hbm.at[pl.ds(base * 2 * D, CH * 2 * D)])
        return 0

    lax.fori_loop(0, NCH, chunk, 0)


def kernel(title_table, text_table, title_ids, token_ids):
    text_z = text_table.at[0].set(0.0)
    text_p = text_z.reshape(-1, PK)
    npad = (-title_table.size) % PK
    title_p = jnp.pad(title_table.reshape(-1), (0, npad)).reshape(-1, PK)
    tb = token_ids.reshape(B // CH, CH, L).transpose(0, 2, 1).reshape(-1)
    rb = tb >> 2
    trids = title_ids >> 2
    flat = _sc_kernel(title_p, text_p, title_ids, trids, tb, rb)
    return flat.reshape(B, 2 * D)


# speedup vs baseline: 9.2440x; 1.2100x over previous
import functools

import jax
import jax.numpy as jnp
from jax import lax
from jax.experimental import pallas as pl
from jax.experimental.pallas import tpu as pltpu
from jax.experimental.pallas import tpu_sc as plsc

B = 16384
L = 20
D = 32
PK = 128

NC, NS = 2, 16
NW = NC * NS
SPW = B // NW
CH = 16
NCH = SPW // CH

RID0 = L * CH
TIX0 = 2 * L * CH
TRI0 = TIX0 + CH
REC = TRI0 + CH

_MESH = plsc.VectorSubcoreMesh(
    core_axis_name="c", subcore_axis_name="s", num_cores=NC, num_subcores=NS)


@functools.partial(
    pl.kernel,
    out_type=jax.ShapeDtypeStruct((B * 2 * D,), jnp.float32),
    mesh=_MESH,
    scratch_types=[
        pltpu.VMEM((REC,), jnp.int32),
        pltpu.VMEM((REC,), jnp.int32),
        pltpu.VMEM((L * CH, PK), jnp.float32),
        pltpu.VMEM((L * CH, PK), jnp.float32),
        pltpu.VMEM((CH, PK), jnp.float32),
        pltpu.VMEM((CH, PK), jnp.float32),
        pltpu.VMEM((SPW * 2 * D,), jnp.float32),
        pltpu.SemaphoreType.DMA,
        pltpu.SemaphoreType.DMA,
        pltpu.SemaphoreType.DMA,
    ],
)
def _sc_kernel(title_hbm, text_hbm, rec_hbm, out_hbm,
               idx0, idx1, rows0, rows1, trow0, trow1, outw,
               isem, gsem, tsem):
    wid = lax.axis_index("s") * NC + lax.axis_index("c")
    cid0 = wid * NCH

    def stage_idx(cidx, ib):
        return pltpu.async_copy(rec_hbm.at[pl.ds(cidx * REC, REC)], ib, isem)

    def wait_idx(cidx, ib):
        pltpu.make_async_copy(
            rec_hbm.at[pl.ds(cidx * REC, REC)], ib, isem).wait()

    def fire_gathers(ib, rb, tb):
        for j in range(L):
            pltpu.async_copy(text_hbm.at[ib.at[pl.ds(RID0 + j * CH, CH)]],
                             rb.at[pl.ds(j * CH, CH)], gsem)
        pltpu.async_copy(title_hbm.at[ib.at[pl.ds(TRI0, CH)]], tb, tsem)

    def wait_gathers(ib, rb, tb):
        for j in range(L):
            pltpu.make_async_copy(
                text_hbm.at[ib.at[pl.ds(RID0 + j * CH, CH)]],
                rb.at[pl.ds(j * CH, CH)], gsem).wait()
        pltpu.make_async_copy(
            title_hbm.at[ib.at[pl.ds(TRI0, CH)]], tb, tsem).wait()

    pltpu.sync_copy(rec_hbm.at[pl.ds(cid0 * REC, REC)], idx0)
    fire_gathers(idx0, rows0, trow0)
    stage_idx(cid0 + 1, idx1)

    bufs = ((idx0, rows0, trow0), (idx1, rows1, trow1))

    def body(cc, _):
        for p in range(2):
            c = cc * 2 + p
            cidx = cid0 + c
            ib, rb, tb = bufs[p]
            ibn, rbn, tbn = bufs[1 - p]

            @pl.when(c + 1 < NCH)
            def _():
                wait_idx(cidx + 1, ibn)
                fire_gathers(ibn, rbn, tbn)

            ivs = [ib[pl.ds(j * CH, 16)] for j in range(L)]
            cnt = jnp.zeros((16,), jnp.float32)
            for iv in ivs:
                cnt = cnt + jnp.where(iv != 0, 1.0, 0.0)
            rvec = 1.0 / jnp.maximum(cnt, 1.0)
            qvecs = [(iv & 3) * D for iv in ivs]
            tqv = (ib[pl.ds(TIX0, 16)] & 3) * D

            @pl.when(c + 2 < NCH)
            def _():
                stage_idx(cidx + 2, ib)

            wait_gathers(ib, rb, tb)

            for i2 in range(16):
                q0 = qvecs[0][i2]
                a0 = rb[i2, pl.ds(q0, 16)]
                a1 = rb[i2, pl.ds(q0 + 16, 16)]
                for j in range(1, L):
                    qj = qvecs[j][i2]
                    a0 = a0 + rb[j * CH + i2, pl.ds(qj, 16)]
                    a1 = a1 + rb[j * CH + i2, pl.ds(qj + 16, 16)]
                r = rvec[i2]
                tq = tqv[i2]
                ob = pl.multiple_of(c * (CH * 2 * D) + i2 * 2 * D, 2 * D)
                outw[pl.ds(ob, 16)] = tb[i2, pl.ds(tq, 16)]
                outw[pl.ds(ob + 16, 16)] = tb[i2, pl.ds(tq + 16, 16)]
                outw[pl.ds(ob + 32, 16)] = a0 * r
                outw[pl.ds(ob + 48, 16)] = a1 * r
        return 0

    lax.fori_loop(0, NCH // 2, body, 0)
    pltpu.sync_copy(outw, out_hbm.at[pl.ds(wid * (SPW * 2 * D), SPW * 2 * D)])


def kernel(title_table, text_table, title_ids, token_ids):
    text_z = text_table.at[0].set(0.0)
    text_p = text_z.reshape(-1, PK)
    npad = (-title_table.size) % PK
    title_p = jnp.pad(title_table.reshape(-1), (0, npad)).reshape(-1, PK)
    tb = token_ids.reshape(B // CH, CH, L).transpose(0, 2, 1).reshape(
        B // CH, L * CH)
    rec = jnp.concatenate(
        [tb, tb >> 2, title_ids.reshape(B // CH, CH),
         (title_ids >> 2).reshape(B // CH, CH)], axis=1).reshape(-1)
    flat = _sc_kernel(title_p, text_p, rec)
    return flat.reshape(B, 2 * D)


# speedup vs baseline: 10.1433x; 1.0973x over previous
import functools

import jax
import jax.numpy as jnp
from jax import lax
from jax.experimental import pallas as pl
from jax.experimental.pallas import tpu as pltpu
from jax.experimental.pallas import tpu_sc as plsc

B = 16384
L = 20
D = 32
PK = 128

NC, NS = 2, 16
NW = NC * NS
SPW = B // NW
CH = 16
NCH = SPW // CH

_MESH = plsc.VectorSubcoreMesh(
    core_axis_name="c", subcore_axis_name="s", num_cores=NC, num_subcores=NS)


@functools.partial(
    pl.kernel,
    out_type=jax.ShapeDtypeStruct((B * 2 * D,), jnp.float32),
    mesh=_MESH,
    scratch_types=[
        pltpu.VMEM((L * CH,), jnp.int32),
        pltpu.VMEM((L * CH,), jnp.int32),
        pltpu.VMEM((CH,), jnp.int32),
        pltpu.VMEM((CH,), jnp.int32),
        pltpu.VMEM((L * CH, PK), jnp.float32),
        pltpu.VMEM((L * CH, PK), jnp.float32),
        pltpu.VMEM((CH, PK), jnp.float32),
        pltpu.VMEM((CH, PK), jnp.float32),
        pltpu.VMEM((SPW * 2 * D,), jnp.float32),
        pltpu.SemaphoreType.DMA,
        pltpu.SemaphoreType.DMA,
        pltpu.SemaphoreType.DMA,
    ],
)
def _sc_kernel(title_hbm, text_hbm, tids_hbm, tb_hbm, out_hbm,
               ids0, ids1, tix0, tix1, rows0, rows1, trow0, trow1, outw,
               isem, gsem, tsem):
    wid = lax.axis_index("s") * NC + lax.axis_index("c")
    cid0 = wid * NCH

    def stage_idx(cidx, ib, xb):
        pltpu.async_copy(tb_hbm.at[pl.ds(cidx * (L * CH), L * CH)], ib, isem)
        pltpu.async_copy(tids_hbm.at[pl.ds(cidx * CH, CH)], xb, isem)

    def wait_idx(cidx, ib, xb):
        pltpu.make_async_copy(
            tb_hbm.at[pl.ds(cidx * (L * CH), L * CH)], ib, isem).wait()
        pltpu.make_async_copy(
            tids_hbm.at[pl.ds(cidx * CH, CH)], xb, isem).wait()

    def fire_gathers(ib, xb, rb, tb):
        for j in range(L):
            pltpu.async_copy(text_hbm.at[ib.at[pl.ds(j * CH, CH)]],
                             rb.at[pl.ds(j * CH, CH)], gsem)
        pltpu.async_copy(title_hbm.at[xb], tb, tsem)

    def wait_gathers(ib, xb, rb, tb):
        for j in range(L):
            pltpu.make_async_copy(text_hbm.at[ib.at[pl.ds(j * CH, CH)]],
                                  rb.at[pl.ds(j * CH, CH)], gsem).wait()
        pltpu.make_async_copy(title_hbm.at[xb], tb, tsem).wait()

    pltpu.sync_copy(tb_hbm.at[pl.ds(cid0 * (L * CH), L * CH)], ids0)
    pltpu.sync_copy(tids_hbm.at[pl.ds(cid0 * CH, CH)], tix0)
    fire_gathers(ids0, tix0, rows0, trow0)
    stage_idx(cid0 + 1, ids1, tix1)

    bufs = ((ids0, tix0, rows0, trow0), (ids1, tix1, rows1, trow1))

    def body(cc, _):
        for p in range(2):
            c = cc * 2 + p
            cidx = cid0 + c
            ib, xb, rb, tb = bufs[p]
            ibn, xbn, rbn, tbn = bufs[1 - p]

            @pl.when(c + 1 < NCH)
            def _():
                wait_idx(cidx + 1, ibn, xbn)
                fire_gathers(ibn, xbn, rbn, tbn)

            cnt = jnp.zeros((16,), jnp.float32)
            for j in range(L):
                iv = ib[pl.ds(j * CH, 16)]
                cnt = cnt + jnp.where(iv != 0, 1.0, 0.0)
            rvec = 1.0 / jnp.maximum(cnt, 1.0)

            @pl.when(c + 2 < NCH)
            def _():
                stage_idx(cidx + 2, ib, xb)

            wait_gathers(ib, xb, rb, tb)

            for i2 in range(16):
                a0 = rb[i2, pl.ds(0, 16)]
                a1 = rb[i2, pl.ds(16, 16)]
                for j in range(1, L):
                    a0 = a0 + rb[j * CH + i2, pl.ds(0, 16)]
                    a1 = a1 + rb[j * CH + i2, pl.ds(16, 16)]
                r = rvec[i2]
                ob = pl.multiple_of(c * (CH * 2 * D) + i2 * 2 * D, 2 * D)
                outw[pl.ds(ob, 16)] = tb[i2, pl.ds(0, 16)]
                outw[pl.ds(ob + 16, 16)] = tb[i2, pl.ds(16, 16)]
                outw[pl.ds(ob + 32, 16)] = a0 * r
                outw[pl.ds(ob + 48, 16)] = a1 * r
        return 0

    lax.fori_loop(0, NCH // 2, body, 0)
    pltpu.sync_copy(outw, out_hbm.at[pl.ds(wid * (SPW * 2 * D), SPW * 2 * D)])


def kernel(title_table, text_table, title_ids, token_ids):
    text_z = text_table.at[0].set(0.0)
    text_p = jnp.pad(text_z, ((0, 0), (0, PK - D)))
    title_p = jnp.pad(title_table, ((0, 7), (0, PK - D)))
    tb = token_ids.reshape(B // CH, CH, L).transpose(0, 2, 1).reshape(-1)
    flat = _sc_kernel(title_p, text_p, title_ids, tb)
    return flat.reshape(B, 2 * D)


# speedup vs baseline: 11.1854x; 1.1027x over previous
import functools

import jax
import jax.numpy as jnp
from jax import lax
from jax.experimental import pallas as pl
from jax.experimental.pallas import tpu as pltpu
from jax.experimental.pallas import tpu_sc as plsc

B = 16384
L = 20
D = 32
PK = 128

NC, NS = 2, 16
NW = NC * NS
SPW = B // NW
CH = 16
NCH = SPW // CH

_MESH = plsc.VectorSubcoreMesh(
    core_axis_name="c", subcore_axis_name="s", num_cores=NC, num_subcores=NS)


@functools.partial(
    pl.kernel,
    out_type=jax.ShapeDtypeStruct((B * 2 * D,), jnp.float32),
    mesh=_MESH,
    compiler_params=pltpu.CompilerParams(needs_layout_passes=False),
    scratch_types=[
        pltpu.VMEM((L * CH,), jnp.int32),
        pltpu.VMEM((L * CH,), jnp.int32),
        pltpu.VMEM((CH,), jnp.int32),
        pltpu.VMEM((CH,), jnp.int32),
        pltpu.VMEM((L * CH, PK), jnp.float32),
        pltpu.VMEM((L * CH, PK), jnp.float32),
        pltpu.VMEM((CH, PK), jnp.float32),
        pltpu.VMEM((CH, PK), jnp.float32),
        pltpu.VMEM((SPW * 2 * D,), jnp.float32),
        pltpu.SemaphoreType.DMA,
        pltpu.SemaphoreType.DMA,
        pltpu.SemaphoreType.DMA,
    ],
)
def _sc_kernel(title_hbm, text_hbm, tids_hbm, tb_hbm, out_hbm,
               ids0, ids1, tix0, tix1, rows0, rows1, trow0, trow1, outw,
               isem, gsem, tsem):
    wid = lax.axis_index("s") * NC + lax.axis_index("c")
    cid0 = wid * NCH

    def stage_idx(cidx, ib, xb):
        pltpu.async_copy(tb_hbm.at[pl.ds(cidx * (L * CH), L * CH)], ib, isem)
        pltpu.async_copy(tids_hbm.at[pl.ds(cidx * CH, CH)], xb, isem)

    def wait_idx(cidx, ib, xb):
        pltpu.make_async_copy(
            tb_hbm.at[pl.ds(cidx * (L * CH), L * CH)], ib, isem).wait()
        pltpu.make_async_copy(
            tids_hbm.at[pl.ds(cidx * CH, CH)], xb, isem).wait()

    def fire_gathers(ib, xb, rb, tb):
        for j in range(L):
            pltpu.async_copy(text_hbm.at[ib.at[pl.ds(j * CH, CH)]],
                             rb.at[pl.ds(j * CH, CH)], gsem)
        pltpu.async_copy(title_hbm.at[xb], tb, tsem)

    def wait_gathers(ib, xb, rb, tb):
        for j in range(L):
            pltpu.make_async_copy(text_hbm.at[ib.at[pl.ds(j * CH, CH)]],
                                  rb.at[pl.ds(j * CH, CH)], gsem).wait()
        pltpu.make_async_copy(title_hbm.at[xb], tb, tsem).wait()

    pltpu.sync_copy(tb_hbm.at[pl.ds(cid0 * (L * CH), L * CH)], ids0)
    pltpu.sync_copy(tids_hbm.at[pl.ds(cid0 * CH, CH)], tix0)
    fire_gathers(ids0, tix0, rows0, trow0)
    stage_idx(cid0 + 1, ids1, tix1)

    bufs = ((ids0, tix0, rows0, trow0), (ids1, tix1, rows1, trow1))

    def body(cc, _):
        for p in range(2):
            c = cc * 2 + p
            cidx = cid0 + c
            ib, xb, rb, tb = bufs[p]
            ibn, xbn, rbn, tbn = bufs[1 - p]

            @pl.when(c + 1 < NCH)
            def _():
                wait_idx(cidx + 1, ibn, xbn)
                fire_gathers(ibn, xbn, rbn, tbn)

            iot = lax.iota(jnp.int32, 16) * L
            cnt = jnp.zeros((16,), jnp.float32)
            for j in range(L):
                iv = plsc.load_gather(ib, [iot + j])
                cnt = cnt + jnp.where(iv != 0, 1.0, 0.0)
            rvec = 1.0 / jnp.maximum(cnt, 1.0)

            @pl.when(c + 2 < NCH)
            def _():
                stage_idx(cidx + 2, ib, xb)

            wait_gathers(ib, xb, rb, tb)

            for i2 in range(16):
                a0 = rb[i2 * L, pl.ds(0, 16)]
                a1 = rb[i2 * L, pl.ds(16, 16)]
                for j in range(1, L):
                    a0 = a0 + rb[i2 * L + j, pl.ds(0, 16)]
                    a1 = a1 + rb[i2 * L + j, pl.ds(16, 16)]
                r = rvec[i2]
                ob = pl.multiple_of(c * (CH * 2 * D) + i2 * 2 * D, 2 * D)
                outw[pl.ds(ob, 16)] = tb[i2, pl.ds(0, 16)]
                outw[pl.ds(ob + 16, 16)] = tb[i2, pl.ds(16, 16)]
                outw[pl.ds(ob + 32, 16)] = a0 * r
                outw[pl.ds(ob + 48, 16)] = a1 * r
        return 0

    lax.fori_loop(0, NCH // 2, body, 0)
    pltpu.sync_copy(outw, out_hbm.at[pl.ds(wid * (SPW * 2 * D), SPW * 2 * D)])


def kernel(title_table, text_table, title_ids, token_ids):
    text_z = text_table.at[0].set(0.0)
    text_p = jnp.pad(text_z, ((0, 0), (0, PK - D)))
    title_p = jnp.pad(title_table, ((0, 7), (0, PK - D)))
    tb = token_ids.reshape(-1)
    flat = _sc_kernel(title_p, text_p, title_ids, tb)
    return flat.reshape(B, 2 * D)
